# baseline (speedup 1.0000x reference)
import jax
import jax.numpy as jnp
from jax import lax
from jax.experimental import pallas as pl
from jax.experimental.pallas import tpu as pltpu
from jax.experimental.pallas import tpu_sc as plsc

NC = 2
NS = 16
NW = NC * NS


def _gather_kernel(table_hbm, idx_hbm, out_hbm, idx_v, rows0, rows1,
                   g0, g1, w0, w1):
    wid = lax.axis_index("s") * NC + lax.axis_index("c")
    n_chunks = idx_hbm.shape[1]
    base = wid * n_chunks

    pltpu.sync_copy(idx_hbm.at[wid], idx_v)

    def gather_start(j, buf, sem):
        pltpu.async_copy(table_hbm.at[idx_v.at[j]], buf, sem)

    def gather_wait(buf, sem):
        pltpu.make_async_copy(table_hbm.at[idx_v.at[0]], buf, sem).wait()

    def write_start(j, buf, sem):
        pltpu.async_copy(buf, out_hbm.at[base + j], sem)

    def write_wait(buf, sem):
        pltpu.make_async_copy(buf, out_hbm.at[base], sem).wait()

    assert n_chunks % 2 == 0 and n_chunks >= 4
    gather_start(0, rows0, g0)
    gather_start(1, rows1, g1)

    @pl.loop(0, n_chunks - 2, step=2)
    def _(jj):
        gather_wait(rows0, g0)
        write_start(jj, rows0, w0)
        gather_wait(rows1, g1)
        write_start(jj + 1, rows1, w1)
        write_wait(rows0, w0)
        gather_start(jj + 2, rows0, g0)
        write_wait(rows1, w1)
        gather_start(jj + 3, rows1, g1)

    jj = n_chunks - 2
    gather_wait(rows0, g0)
    write_start(jj, rows0, w0)
    gather_wait(rows1, g1)
    write_start(jj + 1, rows1, w1)
    write_wait(rows0, w0)
    write_wait(rows1, w1)


def kernel(input_ids, token_emb):
    B, T = input_ids.shape
    V, D = token_emb.shape
    assert B % NW == 0
    n_chunks = B // NW
    TP = (T + 7) // 8 * 8

    idx = input_ids.astype(jnp.int32)
    if TP != T:
        idx = jnp.pad(idx, ((0, 0), (0, TP - T)))
    idx = idx.reshape(NW, n_chunks, TP)

    mesh = plsc.VectorSubcoreMesh(core_axis_name="c", subcore_axis_name="s")
    k = pl.kernel(
        _gather_kernel,
        out_type=jax.ShapeDtypeStruct((B, TP, D), jnp.float32),
        mesh=mesh,
        scratch_types=[
            pltpu.VMEM((n_chunks, TP), jnp.int32),
            pltpu.VMEM((TP, D), jnp.float32),
            pltpu.VMEM((TP, D), jnp.float32),
            pltpu.SemaphoreType.DMA,
            pltpu.SemaphoreType.DMA,
            pltpu.SemaphoreType.DMA,
            pltpu.SemaphoreType.DMA,
        ],
    )
    out = k(token_emb, idx)
    return out[:, :T, :]


# speedup vs baseline: 2.4984x; 2.4984x over previous
import jax
import jax.numpy as jnp
from jax import lax
from jax.experimental import pallas as pl
from jax.experimental.pallas import tpu as pltpu
from jax.experimental.pallas import tpu_sc as plsc

NC = 2
NS = 16
NW = NC * NS


def _gather_kernel(table_hbm, idx_hbm, out_hbm, idx_v, rows0, rows1,
                   g0, g1, w0, w1):
    wid = lax.axis_index("s") * NC + lax.axis_index("c")
    n_chunks = idx_hbm.shape[1]
    base = wid * n_chunks

    pltpu.sync_copy(idx_hbm.at[wid], idx_v)

    def gather_start(j, buf, sem):
        pltpu.async_copy(table_hbm.at[idx_v.at[j]], buf, sem)

    def gather_wait(buf, sem):
        pltpu.make_async_copy(table_hbm.at[idx_v.at[0]], buf, sem).wait()

    def write_start(j, buf, sem):
        pltpu.async_copy(buf, out_hbm.at[base + j], sem)

    def write_wait(buf, sem):
        pltpu.make_async_copy(buf, out_hbm.at[base], sem).wait()

    assert n_chunks % 2 == 0 and n_chunks >= 4
    gather_start(0, rows0, g0)
    gather_start(1, rows1, g1)

    @pl.loop(0, n_chunks - 2, step=2)
    def _(jj):
        gather_wait(rows0, g0)
        write_start(jj, rows0, w0)
        gather_wait(rows1, g1)
        write_start(jj + 1, rows1, w1)
        write_wait(rows0, w0)
        gather_start(jj + 2, rows0, g0)
        write_wait(rows1, w1)
        gather_start(jj + 3, rows1, g1)

    jj = n_chunks - 2
    gather_wait(rows0, g0)
    write_start(jj, rows0, w0)
    gather_wait(rows1, g1)
    write_start(jj + 1, rows1, w1)
    write_wait(rows0, w0)
    write_wait(rows1, w1)


def kernel(input_ids, token_emb):
    B, T = input_ids.shape
    V, D = token_emb.shape
    assert B % NW == 0
    n_chunks = B // NW
    TP = (T + 7) // 8 * 8

    idx = input_ids.astype(jnp.int32)
    if TP != T:
        idx = jnp.concatenate([idx, idx[:, : TP - T]], axis=1)
    idx = idx.reshape(NW, n_chunks, TP)

    mesh = plsc.VectorSubcoreMesh(core_axis_name="c", subcore_axis_name="s")
    k = pl.kernel(
        _gather_kernel,
        out_type=jax.ShapeDtypeStruct((B, TP, D), jnp.float32),
        mesh=mesh,
        scratch_types=[
            pltpu.VMEM((n_chunks, TP), jnp.int32),
            pltpu.VMEM((TP, D), jnp.float32),
            pltpu.VMEM((TP, D), jnp.float32),
            pltpu.SemaphoreType.DMA,
            pltpu.SemaphoreType.DMA,
            pltpu.SemaphoreType.DMA,
            pltpu.SemaphoreType.DMA,
        ],
    )
    out = k(token_emb, idx)
    return out[:, :T, :]
